# baseline (speedup 1.0000x reference)
import functools

import jax
import jax.numpy as jnp
from jax import lax
from jax.experimental import pallas as pl
from jax.experimental.pallas import tpu as pltpu
from jax.experimental.pallas import tpu_sc as plsc

INF = 2**31 - 1
NC = 2
NS = 16
ROWS = 1000
FQ = 10240
RCH = 64


def _k1_body(feats_ref, wc4_ref, bc_ref, wm_ref, bm_ref,
             z0_ref, z1_ref, z2_ref, z3_ref, init_ref):
    f = feats_ref[0]
    z = jnp.dot(f, wc4_ref[...], preferred_element_type=jnp.float32)
    c = f.shape[1]
    z0_ref[...] = (z[:, 0 * c:1 * c] + bc_ref[...])[None]
    z1_ref[...] = z[:, 1 * c:2 * c][None]
    z2_ref[...] = z[:, 2 * c:3 * c][None]
    z3_ref[...] = z[:, 3 * c:4 * c][None]
    s = jnp.sum(f * wm_ref[...], axis=1, keepdims=True) + bm_ref[...]
    init_ref[...] = jax.nn.sigmoid(s)[None]


def _k1(feats, Wc4, bc2, wm2, bm2):
    bn, fn, c = feats.shape
    grid = (bn, fn // ROWS)
    zspec = pl.BlockSpec((1, ROWS, c), lambda b, j: (b, j, 0))
    zshape = jax.ShapeDtypeStruct((bn, FQ, c), jnp.float32)
    return pl.pallas_call(
        _k1_body,
        grid=grid,
        in_specs=[zspec,
                  pl.BlockSpec((c, 4 * c), lambda b, j: (0, 0)),
                  pl.BlockSpec((1, c), lambda b, j: (0, 0)),
                  pl.BlockSpec((1, c), lambda b, j: (0, 0)),
                  pl.BlockSpec((1, 1), lambda b, j: (0, 0))],
        out_specs=[zspec, zspec, zspec, zspec,
                   pl.BlockSpec((1, ROWS, 1), lambda b, j: (b, j, 0))],
        out_shape=[zshape, zshape, zshape, zshape,
                   jax.ShapeDtypeStruct((bn, FQ, 1), jnp.float32)],
    )(feats, Wc4, bc2, wm2, bm2)


def _mega_body(z0f, z1f, z2f, z3f, a0f, a1f, a2f, a0q, a1q, a2q,
               initf, qih, dih, tgh, wmh,
               bfh, prh, sch, dph,
               I0, I1, I2, R0, R1, R2, R3, PRC, WMV,
               A0, A1, A2, PR, SCR, DQ, QU, TG,
               SG0, SG1, SG2, SO):
    cx = lax.axis_index("c")
    sx = lax.axis_index("s")
    iota16 = lax.iota(jnp.int32, 16)
    base = cx * FQ + sx * (FQ // NS)

    pltpu.sync_copy(wmh, WMV)
    wmv = tuple(WMV[pl.ds(16 * t, 16)] for t in range(8))
    bmv = WMV[pl.ds(128, 16)]

    half = jnp.float32(0.5)
    three_half = jnp.float32(1.5)
    inv_c = jnp.float32(1.0 / 128.0)
    magic = jnp.int32(0x5F3759DF)

    def row_body(r, carry):
        h = []
        for t in range(8):
            cs = pl.ds(16 * t, 16)
            h.append(R0[r, cs] + R1[r, cs] + R2[r, cs] + R3[r, cs])
        ssum = h[0]
        for t in range(1, 8):
            ssum = ssum + h[t]
        sqsum = h[0] * h[0]
        for t in range(1, 8):
            sqsum = sqsum + h[t] * h[t]
        s1 = jnp.sum(ssum)
        s2 = jnp.sum(sqsum)
        mu = s1 * inv_c
        var = s2 * inv_c - mu * mu + jnp.float32(1e-5)
        vv = jnp.zeros((16,), jnp.float32) + var
        y = plsc.bitcast(magic - (plsc.bitcast(vv, jnp.int32) >> 1),
                         jnp.float32)
        for _ in range(3):
            y = y * (three_half - half * vv * y * y)
        muv = jnp.zeros((16,), jnp.float32) + mu
        psum = jnp.zeros((16,), jnp.float32)
        for t in range(8):
            cs = pl.ds(16 * t, 16)
            bf_t = (h[t] - muv) * y
            R0[r, cs] = bf_t
            psum = psum + bf_t * carry[t]
        pv = jnp.zeros((16,), jnp.float32) + jnp.sum(psum)
        sig = 1.0 / (1.0 + jnp.exp(-(pv + carry[8])))
        plsc.store_scatter(PRC, [jnp.zeros((16,), jnp.int32) + r], sig,
                           mask=iota16 == (r & 15))
        return carry

    so_desc = None
    for j in range(FQ // NS // RCH):
        st = base + j * RCH
        pltpu.sync_copy(a0f.at[pl.ds(st, RCH)], I0)
        pltpu.sync_copy(a1f.at[pl.ds(st, RCH)], I1)
        pltpu.sync_copy(a2f.at[pl.ds(st, RCH)], I2)
        g1 = pltpu.async_copy(z1f.at[I0], R1, SG0)
        g2 = pltpu.async_copy(z2f.at[I1], R2, SG1)
        g3 = pltpu.async_copy(z3f.at[I2], R3, SG2)
        if so_desc is not None:
            so_desc.wait()
        pltpu.sync_copy(z0f.at[pl.ds(st, RCH)], R0)
        g1.wait()
        g2.wait()
        g3.wait()
        lax.fori_loop(0, RCH, row_body, wmv + (bmv,))
        so_desc = pltpu.async_copy(R0, bfh.at[pl.ds(st, RCH)], SO)
        pltpu.sync_copy(PRC, prh.at[pl.ds(st, RCH)])
    so_desc.wait()
    plsc.subcore_barrier()

    bn = sch.shape[0]
    active = (sx == 0) & (cx < bn)
    b = jnp.minimum(cx, bn - 1)

    @pl.when(active)
    def _():
        pltpu.sync_copy(a0q.at[b], A0)
        pltpu.sync_copy(a1q.at[b], A1)
        pltpu.sync_copy(a2q.at[b], A2)
        pltpu.sync_copy(prh.at[pl.ds(b * FQ, FQ)], PR)
        pltpu.sync_copy(initf.at[pl.ds(b * FQ, FQ)], SCR)
        pltpu.sync_copy(qih.at[b], QU.at[pl.ds(0, FQ)])
        pltpu.sync_copy(dih.at[b], DQ)
        pltpu.sync_copy(tgh, TG)

    def step(carry):
        base_q, hi, tl, level, tok = carry
        new_lvl = base_q >= hi
        level = jnp.where(new_lvl, level + 1, level)
        base_q = jnp.where(new_lvl, hi, base_q)
        hi = jnp.where(new_lvl, tl, hi)
        lanes = base_q + iota16
        m = lanes < hi
        fv = plsc.load_gather(QU, [jnp.where(m, lanes, 0)])
        f = jnp.where(m, fv, 0)
        a0v = plsc.load_gather(A0, [f])
        a1v = plsc.load_gather(A1, [f])
        a2v = plsc.load_gather(A2, [f])
        d0 = plsc.load_gather(DQ, [a0v])
        d1 = plsc.load_gather(DQ, [a1v])
        d2 = plsc.load_gather(DQ, [a2v])
        s0 = plsc.load_gather(SCR, [a0v])
        s1 = plsc.load_gather(SCR, [a1v])
        s2 = plsc.load_gather(SCR, [a2v])
        v0 = d0 < level
        v1 = d1 < level
        v2 = d2 < level
        neg = jnp.float32(-1e30)
        nb = jnp.maximum(jnp.maximum(jnp.where(v0, s0, neg),
                                     jnp.where(v1, s1, neg)),
                         jnp.where(v2, s2, neg))
        has = v0 | v1 | v2
        nbv = jnp.where(has, nb, jnp.float32(1.0))
        pf = plsc.load_gather(PR, [f])
        sf = plsc.load_gather(SCR, [f])
        val = jnp.minimum(jnp.maximum(pf, sf), nbv)
        plsc.store_scatter(SCR, [f], val, mask=m)
        dnew = jnp.zeros((16,), jnp.int32) + (level + 1)

        def expand(av, dv, tl, tok):
            cand = m & (dv == INF)
            tokv = tok + iota16
            plsc.store_scatter(TG, [av], tokv, mask=cand)
            tt = plsc.load_gather(TG, [av])
            win = cand & (tt == tokv)
            plsc.store_scatter(DQ, [av], dnew, mask=cand)
            plsc.store_compressed(QU.at[pl.ds(tl, 16)], av, mask=win)
            cnt = plsc.all_reduce_population_count(win)[0]
            return tl + cnt, tok + 16

        tl, tok = expand(a0v, d0, tl, tok)
        tl, tok = expand(a1v, d1, tl, tok)
        tl, tok = expand(a2v, d2, tl, tok)
        return base_q + 16, hi, tl, level, tok

    def not_done(carry):
        base_q, hi, tl, _, _ = carry
        return (base_q < hi) | (hi < tl)

    one_if = jnp.where(active, jnp.int32(1), jnp.int32(0))
    lax.while_loop(not_done, step,
                   (jnp.int32(0), one_if, one_if, jnp.int32(0),
                    jnp.int32(0)))

    @pl.when(active)
    def _():
        pltpu.sync_copy(SCR, sch.at[b])
        pltpu.sync_copy(DQ, dph.at[b])


def _mega(z0f, z1f, z2f, z3f, a0f, a1f, a2f, a0q, a1q, a2q,
          initf, qi, di, tgi, wmh):
    bn = a0q.shape[0]
    c = z0f.shape[1]
    mesh = plsc.VectorSubcoreMesh(core_axis_name="c", subcore_axis_name="s")
    run = functools.partial(
        pl.kernel,
        out_type=[jax.ShapeDtypeStruct((bn * FQ, c), jnp.float32),
                  jax.ShapeDtypeStruct((bn * FQ,), jnp.float32),
                  jax.ShapeDtypeStruct((bn, FQ), jnp.float32),
                  jax.ShapeDtypeStruct((bn, FQ), jnp.int32)],
        mesh=mesh,
        scratch_types=[pltpu.VMEM((RCH,), jnp.int32)] * 3
        + [pltpu.VMEM((RCH, c), jnp.float32)] * 4
        + [pltpu.VMEM((RCH,), jnp.float32)]
        + [pltpu.VMEM((144,), jnp.float32)]
        + [pltpu.VMEM((FQ,), jnp.int32)] * 3
        + [pltpu.VMEM((FQ,), jnp.float32)] * 2
        + [pltpu.VMEM((FQ,), jnp.int32),
           pltpu.VMEM((FQ + 16,), jnp.int32),
           pltpu.VMEM((FQ,), jnp.int32)]
        + [pltpu.SemaphoreType.DMA] * 4,
        compiler_params=pltpu.CompilerParams(needs_layout_passes=False),
    )(_mega_body)
    return run(z0f, z1f, z2f, z3f, a0f, a1f, a2f, a0q, a1q, a2q,
               initf, qi, di, tgi, wmh)


def _k5_body(feats_ref, bf_ref, depth_ref, out_ref):
    reached = depth_ref[...] != INF
    out_ref[...] = jnp.where(reached, bf_ref[...], feats_ref[...])


def _k5(feats, bf3, depth3):
    bn, fn, c = feats.shape
    grid = (bn, fn // ROWS)
    fspec = pl.BlockSpec((1, ROWS, c), lambda b, j: (b, j, 0))
    return pl.pallas_call(
        _k5_body,
        grid=grid,
        in_specs=[fspec, fspec,
                  pl.BlockSpec((1, ROWS, 1), lambda b, j: (b, j, 0))],
        out_specs=fspec,
        out_shape=jax.ShapeDtypeStruct((bn, fn, c), jnp.float32),
    )(feats, bf3, depth3)


def kernel(x, face_adj, anchors, Wc, bc, Wm, bm):
    bn, cn, fn = x.shape

    feats = jnp.transpose(x, (0, 2, 1))

    Wc4 = jnp.concatenate([Wc[k * cn:(k + 1) * cn] for k in range(4)],
                          axis=1)
    bc2 = bc.reshape(1, cn)
    wmh = jnp.concatenate(
        [Wm.reshape(cn), jnp.full((16,), bm[0], jnp.float32)])

    padq = ((0, 0), (0, FQ - fn))
    a0q = jnp.pad(face_adj[:, :, 0], padq)
    a1q = jnp.pad(face_adj[:, :, 1], padq)
    a2q = jnp.pad(face_adj[:, :, 2], padq)
    offs = (jnp.arange(bn, dtype=jnp.int32) * FQ)[:, None]
    a0f = (a0q + offs).reshape(bn * FQ)
    a1f = (a1q + offs).reshape(bn * FQ)
    a2f = (a2q + offs).reshape(bn * FQ)

    qi = jnp.zeros((bn, FQ), jnp.int32).at[:, 0].set(anchors)
    di = jnp.full((bn, FQ), INF, jnp.int32).at[
        jnp.arange(bn), anchors].set(-1)
    tgi = jnp.full((FQ,), -1, jnp.int32)

    wm2 = Wm.reshape(1, cn)
    bm2 = bm.reshape(1, 1)
    z0, z1, z2, z3, initsc = _k1(feats, Wc4, bc2, wm2, bm2)
    initf = initsc.reshape(bn * FQ)
    z0f = z0.reshape(bn * FQ, cn)
    z1f = z1.reshape(bn * FQ, cn)
    z2f = z2.reshape(bn * FQ, cn)
    z3f = z3.reshape(bn * FQ, cn)
    bf, pred, scores, depth = _mega(z0f, z1f, z2f, z3f, a0f, a1f, a2f,
                                    a0q, a1q, a2q, initf, qi, di, tgi, wmh)
    bf3 = bf.reshape(bn, FQ, cn)
    depth3 = depth.reshape(bn, FQ, 1)
    final_features = _k5(feats, bf3, depth3)
    final_scores = scores[:, :fn].reshape(bn, fn, 1)
    return final_features, final_scores


# speedup vs baseline: 1.0386x; 1.0386x over previous
import functools

import jax
import jax.numpy as jnp
from jax import lax
from jax.experimental import pallas as pl
from jax.experimental.pallas import tpu as pltpu
from jax.experimental.pallas import tpu_sc as plsc

INF = 2**31 - 1
NC = 2
NS = 16
ROWS = 1000
CHUNK = 128


def _k2_body(src, a0, a1, a2, g1, g2, g3, *scr):
    I = scr[0:6]
    R = scr[6:12]
    SI = scr[12:18]
    SG = scr[18:24]
    SO = scr[24:30]
    wid = lax.axis_index("s") * NC + lax.axis_index("c")
    npad = g1.shape[0]
    per_w = npad // (NC * NS)
    nchunk = per_w // CHUNK
    gs = (g1, g2, g3)
    adr = (a0, a1, a2)

    def idx_start(j, s):
        st = wid * per_w + j * CHUNK
        return [pltpu.async_copy(adr[k].at[pl.ds(st, CHUNK)], I[s * 3 + k],
                                 SI[s * 3 + k]) for k in range(3)]

    def gather_start(s):
        return [pltpu.async_copy(src.at[I[s * 3 + k]], R[s * 3 + k],
                                 SG[s * 3 + k]) for k in range(3)]

    def out_start(j, s):
        st = wid * per_w + j * CHUNK
        return [pltpu.async_copy(R[s * 3 + k], gs[k].at[pl.ds(st, CHUNK)],
                                 SO[s * 3 + k]) for k in range(3)]

    idesc = {0: idx_start(0, 0)}
    gdesc = {}
    odesc = {}
    for j in range(nchunk):
        s = j % 2
        for d in idesc[j]:
            d.wait()
        if j >= 2:
            for d in odesc[j - 2]:
                d.wait()
        gdesc[j] = gather_start(s)
        if j >= 1:
            for d in gdesc[j - 1]:
                d.wait()
            odesc[j - 1] = out_start(j - 1, 1 - s)
        if j + 1 < nchunk:
            idesc[j + 1] = idx_start(j + 1, 1 - s)
    for d in gdesc[nchunk - 1]:
        d.wait()
    odesc[nchunk - 1] = out_start(nchunk - 1, (nchunk - 1) % 2)
    for j in (nchunk - 2, nchunk - 1):
        for d in odesc[j]:
            d.wait()


def _k2(featsN, a0f, a1f, a2f, npad):
    n, c = featsN.shape
    mesh = plsc.VectorSubcoreMesh(core_axis_name="c", subcore_axis_name="s")
    out = jax.ShapeDtypeStruct((npad, c), jnp.float32)
    run = functools.partial(
        pl.kernel,
        out_type=[out, out, out],
        mesh=mesh,
        scratch_types=[pltpu.VMEM((CHUNK,), jnp.int32)] * 6
        + [pltpu.VMEM((CHUNK, c), jnp.float32)] * 6
        + [pltpu.SemaphoreType.DMA] * 18,
    )(_k2_body)
    return run(featsN, a0f, a1f, a2f)


def _k3_body(f_ref, g1_ref, g2_ref, g3_ref, wc_ref, bc_ref, wm_ref, bm_ref,
             bf_ref, pred_ref, init_ref):
    f = f_ref[...]
    h4 = jnp.concatenate([f, g1_ref[...], g2_ref[...], g3_ref[...]], axis=1)
    h = jnp.dot(h4, wc_ref[...], preferred_element_type=jnp.float32)
    h = h + bc_ref[...]
    mu = jnp.mean(h, axis=1, keepdims=True)
    d = h - mu
    var = jnp.mean(d * d, axis=1, keepdims=True)
    bf = d * lax.rsqrt(var + 1e-5)
    bf_ref[...] = bf
    wm = wm_ref[...]
    bm = bm_ref[...]
    pred_ref[...] = jax.nn.sigmoid(
        jnp.sum(bf * wm, axis=1, keepdims=True) + bm)
    init_ref[...] = jax.nn.sigmoid(
        jnp.sum(f * wm, axis=1, keepdims=True) + bm)


def _k3(featsN, g1, g2, g3, wcb, bc2, wm2, bm2):
    n, c = featsN.shape
    grid = (n // ROWS,)
    fspec = pl.BlockSpec((ROWS, c), lambda i: (i, 0))
    cspec = pl.BlockSpec((ROWS, 1), lambda i: (i, 0))
    return pl.pallas_call(
        _k3_body,
        grid=grid,
        in_specs=[fspec, fspec, fspec, fspec,
                  pl.BlockSpec((4 * c, c), lambda i: (0, 0)),
                  pl.BlockSpec((1, c), lambda i: (0, 0)),
                  pl.BlockSpec((1, c), lambda i: (0, 0)),
                  pl.BlockSpec((1, 1), lambda i: (0, 0))],
        out_specs=[fspec, cspec, cspec],
        out_shape=[jax.ShapeDtypeStruct((n, c), jnp.float32),
                   jax.ShapeDtypeStruct((n, 1), jnp.float32),
                   jax.ShapeDtypeStruct((n, 1), jnp.float32)],
    )(featsN, g1, g2, g3, wcb, bc2, wm2, bm2)


def _k4_body(a0h, a1h, a2h, prh, inh, qih, dih, tgh, sch, dph,
             A0, A1, A2, PR, SCR, DQ, QU, TG):
    bn = a0h.shape[0]
    fn = a0h.shape[1]
    wid = lax.axis_index("s") * NC + lax.axis_index("c")
    active = wid < bn
    b = jnp.minimum(wid, bn - 1)

    @pl.when(active)
    def _():
        pltpu.sync_copy(a0h.at[b], A0)
        pltpu.sync_copy(a1h.at[b], A1)
        pltpu.sync_copy(a2h.at[b], A2)
        pltpu.sync_copy(prh.at[b], PR)
        pltpu.sync_copy(inh.at[b], SCR)
        pltpu.sync_copy(qih.at[b], QU.at[pl.ds(0, fn)])
        pltpu.sync_copy(dih.at[b], DQ)
        pltpu.sync_copy(tgh, TG)

    iota16 = lax.iota(jnp.int32, 16)

    def step(carry):
        base, hi, tl, level, tok = carry
        new_lvl = base >= hi
        level = jnp.where(new_lvl, level + 1, level)
        base = jnp.where(new_lvl, hi, base)
        hi = jnp.where(new_lvl, tl, hi)
        lanes = base + iota16
        m = lanes < hi
        fv = plsc.load_gather(QU, [jnp.where(m, lanes, 0)])
        f = jnp.where(m, fv, 0)
        a0v = plsc.load_gather(A0, [f])
        a1v = plsc.load_gather(A1, [f])
        a2v = plsc.load_gather(A2, [f])
        d0 = plsc.load_gather(DQ, [a0v])
        d1 = plsc.load_gather(DQ, [a1v])
        d2 = plsc.load_gather(DQ, [a2v])
        s0 = plsc.load_gather(SCR, [a0v])
        s1 = plsc.load_gather(SCR, [a1v])
        s2 = plsc.load_gather(SCR, [a2v])
        v0 = d0 < level
        v1 = d1 < level
        v2 = d2 < level
        neg = jnp.float32(-1e30)
        nb = jnp.maximum(jnp.maximum(jnp.where(v0, s0, neg),
                                     jnp.where(v1, s1, neg)),
                         jnp.where(v2, s2, neg))
        has = v0 | v1 | v2
        nbv = jnp.where(has, nb, jnp.float32(1.0))
        pf = plsc.load_gather(PR, [f])
        sf = plsc.load_gather(SCR, [f])
        val = jnp.minimum(jnp.maximum(pf, sf), nbv)
        plsc.store_scatter(SCR, [f], val, mask=m)
        dnew = jnp.zeros((16,), jnp.int32) + (level + 1)

        def expand(av, dv, tl, tok):
            cand = m & (dv == INF)
            tokv = tok + iota16
            plsc.store_scatter(TG, [av], tokv, mask=cand)
            tt = plsc.load_gather(TG, [av])
            win = cand & (tt == tokv)
            plsc.store_scatter(DQ, [av], dnew, mask=cand)
            plsc.store_compressed(QU.at[pl.ds(tl, 16)], av, mask=win)
            cnt = plsc.all_reduce_population_count(win)[0]
            return tl + cnt, tok + 16

        tl, tok = expand(a0v, d0, tl, tok)
        tl, tok = expand(a1v, d1, tl, tok)
        tl, tok = expand(a2v, d2, tl, tok)
        return base + 16, hi, tl, level, tok

    def not_done(carry):
        base, hi, tl, _, _ = carry
        return (base < hi) | (hi < tl)

    one_if = jnp.where(active, jnp.int32(1), jnp.int32(0))
    lax.while_loop(not_done, step,
                   (jnp.int32(0), one_if, one_if, jnp.int32(0),
                    jnp.int32(0)))

    @pl.when(active)
    def _():
        pltpu.sync_copy(SCR, sch.at[b])
        pltpu.sync_copy(DQ, dph.at[b])


def _k4(a0l, a1l, a2l, pred2, init2, qi, di, tgi):
    bn, fn = a0l.shape
    mesh = plsc.VectorSubcoreMesh(core_axis_name="c", subcore_axis_name="s")
    run = functools.partial(
        pl.kernel,
        out_type=[jax.ShapeDtypeStruct((bn, fn), jnp.float32),
                  jax.ShapeDtypeStruct((bn, fn), jnp.int32)],
        mesh=mesh,
        scratch_types=[pltpu.VMEM((fn,), jnp.int32)] * 3
        + [pltpu.VMEM((fn,), jnp.float32)] * 2
        + [pltpu.VMEM((fn,), jnp.int32),
           pltpu.VMEM((fn + 16,), jnp.int32),
           pltpu.VMEM((fn,), jnp.int32)],
        compiler_params=pltpu.CompilerParams(needs_layout_passes=False),
    )(_k4_body)
    return run(a0l, a1l, a2l, pred2, init2, qi, di, tgi)


def _k5_body(feats_ref, bf_ref, depth_ref, out_ref):
    reached = depth_ref[...] != INF
    out_ref[...] = jnp.where(reached, bf_ref[...], feats_ref[...])


def _k5(featsN, bf, depthN):
    n, c = featsN.shape
    grid = (n // ROWS,)
    zspec = pl.BlockSpec((ROWS, c), lambda i: (i, 0))
    return pl.pallas_call(
        _k5_body,
        grid=grid,
        in_specs=[zspec, zspec, pl.BlockSpec((ROWS, 1), lambda i: (i, 0))],
        out_specs=zspec,
        out_shape=jax.ShapeDtypeStruct((n, c), jnp.float32),
    )(featsN, bf, depthN)


def kernel(x, face_adj, anchors, Wc, bc, Wm, bm):
    bn, cn, fn = x.shape
    n = bn * fn
    gran = NC * NS * CHUNK
    npad = ((n + gran - 1) // gran) * gran

    feats = jnp.transpose(x, (0, 2, 1))
    featsN = feats.reshape(n, cn)
    wcb = Wc
    wm2 = Wm.reshape(1, cn)
    bm2 = bm.reshape(1, 1)
    bc2 = bc.reshape(1, cn)

    a0l = face_adj[:, :, 0]
    a1l = face_adj[:, :, 1]
    a2l = face_adj[:, :, 2]
    offs = (jnp.arange(bn, dtype=jnp.int32) * fn)[:, None]
    a0f = jnp.pad((a0l + offs).reshape(n), (0, npad - n))
    a1f = jnp.pad((a1l + offs).reshape(n), (0, npad - n))
    a2f = jnp.pad((a2l + offs).reshape(n), (0, npad - n))

    fq = ((fn + 127) // 128) * 128
    padq = ((0, 0), (0, fq - fn))
    a0q = jnp.pad(a0l, padq)
    a1q = jnp.pad(a1l, padq)
    a2q = jnp.pad(a2l, padq)
    qi = jnp.zeros((bn, fq), jnp.int32).at[:, 0].set(anchors)
    di = jnp.full((bn, fq), INF, jnp.int32).at[
        jnp.arange(bn), anchors].set(-1)
    tgi = jnp.full((fq,), -1, jnp.int32)

    g1, g2, g3 = _k2(featsN, a0f, a1f, a2f, npad)
    bf, pred, init = _k3(featsN, g1, g2, g3, wcb, bc2, wm2, bm2)
    predq = jnp.pad(pred.reshape(bn, fn), padq)
    initq = jnp.pad(init.reshape(bn, fn), padq)
    scores, depth = _k4(a0q, a1q, a2q, predq, initq, qi, di, tgi)
    outfeat = _k5(featsN, bf, depth[:, :fn].reshape(n, 1))

    final_features = outfeat.reshape(bn, fn, cn)
    final_scores = scores[:, :fn].reshape(bn, fn, 1)
    return final_features, final_scores


# speedup vs baseline: 1.1578x; 1.1147x over previous
import functools

import jax
import jax.numpy as jnp
from jax import lax
from jax.experimental import pallas as pl
from jax.experimental.pallas import tpu as pltpu
from jax.experimental.pallas import tpu_sc as plsc

INF = 2**31 - 1
NC = 2
NS = 16
ROWS = 1000
FQ = 10240
RCH = 32


def _k1_body(feats_ref, wc4_ref, bc_ref, wm_ref, bm_ref,
             z0_ref, z1_ref, z2_ref, z3_ref, init_ref):
    f = feats_ref[0]
    z = jnp.dot(f, wc4_ref[...], preferred_element_type=jnp.float32)
    c = f.shape[1]
    z0_ref[...] = (z[:, 0 * c:1 * c] + bc_ref[...])[None]
    z1_ref[...] = z[:, 1 * c:2 * c][None]
    z2_ref[...] = z[:, 2 * c:3 * c][None]
    z3_ref[...] = z[:, 3 * c:4 * c][None]
    s = jnp.sum(f * wm_ref[...], axis=1, keepdims=True) + bm_ref[...]
    init_ref[...] = jax.nn.sigmoid(s)[None]


def _k1(feats, Wc4, bc2, wm2, bm2):
    bn, fn, c = feats.shape
    grid = (bn, fn // ROWS)
    zspec = pl.BlockSpec((1, ROWS, c), lambda b, j: (b, j, 0))
    zshape = jax.ShapeDtypeStruct((bn, FQ, c), jnp.float32)
    return pl.pallas_call(
        _k1_body,
        grid=grid,
        in_specs=[zspec,
                  pl.BlockSpec((c, 4 * c), lambda b, j: (0, 0)),
                  pl.BlockSpec((1, c), lambda b, j: (0, 0)),
                  pl.BlockSpec((1, c), lambda b, j: (0, 0)),
                  pl.BlockSpec((1, 1), lambda b, j: (0, 0))],
        out_specs=[zspec, zspec, zspec, zspec,
                   pl.BlockSpec((1, ROWS, 1), lambda b, j: (b, j, 0))],
        out_shape=[zshape, zshape, zshape, zshape,
                   jax.ShapeDtypeStruct((bn, FQ, 1), jnp.float32)],
    )(feats, Wc4, bc2, wm2, bm2)


def _mega_body(z0f, z1f, z2f, z3f, a0f, a1f, a2f, a0q, a1q, a2q,
               initf, qih, dih, tgh, wmh,
               bfh, prh, sch, dph, *scr):
    I = scr[0:6]
    R0 = scr[6:8]
    RG = scr[8:14]
    PRC = scr[14]
    WMV = scr[15]
    A0, A1, A2, PR, SCR, DQ, QU, TG = scr[16:24]
    SI = scr[24:30]
    SZ = scr[30:32]
    SG = scr[32:38]
    SO = scr[38:40]
    cx = lax.axis_index("c")
    sx = lax.axis_index("s")
    iota16 = lax.iota(jnp.int32, 16)
    base = cx * FQ + sx * (FQ // NS)

    pltpu.sync_copy(wmh, WMV)
    wmv = tuple(WMV[pl.ds(16 * t, 16)] for t in range(8))
    bmv = WMV[pl.ds(128, 16)]

    half = jnp.float32(0.5)
    three_half = jnp.float32(1.5)
    inv_c = jnp.float32(1.0 / 128.0)
    magic = jnp.int32(0x5F3759DF)

    def row_loop(s):
        r0, rg1, rg2, rg3 = R0[s], RG[s * 3], RG[s * 3 + 1], RG[s * 3 + 2]

        def row_body(r, carry):
            h = []
            for t in range(8):
                cs = pl.ds(16 * t, 16)
                h.append(r0[r, cs] + rg1[r, cs] + rg2[r, cs] + rg3[r, cs])
            ssum = h[0]
            for t in range(1, 8):
                ssum = ssum + h[t]
            sqsum = h[0] * h[0]
            for t in range(1, 8):
                sqsum = sqsum + h[t] * h[t]
            s1 = jnp.sum(ssum)
            s2 = jnp.sum(sqsum)
            mu = s1 * inv_c
            var = s2 * inv_c - mu * mu + jnp.float32(1e-5)
            vv = jnp.zeros((16,), jnp.float32) + var
            y = plsc.bitcast(magic - (plsc.bitcast(vv, jnp.int32) >> 1),
                             jnp.float32)
            for _ in range(3):
                y = y * (three_half - half * vv * y * y)
            muv = jnp.zeros((16,), jnp.float32) + mu
            psum = jnp.zeros((16,), jnp.float32)
            for t in range(8):
                cs = pl.ds(16 * t, 16)
                bf_t = (h[t] - muv) * y
                r0[r, cs] = bf_t
                psum = psum + bf_t * carry[t]
            pv = jnp.zeros((16,), jnp.float32) + jnp.sum(psum)
            sig = 1.0 / (1.0 + jnp.exp(-(pv + carry[8])))
            plsc.store_scatter(PRC, [jnp.zeros((16,), jnp.int32) + r], sig,
                               mask=iota16 == (r & 15))
            return carry

        lax.fori_loop(0, RCH, row_body, wmv + (bmv,))

    nch = FQ // NS // RCH
    adrs = (a0f, a1f, a2f)
    zg = (z1f, z2f, z3f)

    def idx_start(j, s):
        st = base + j * RCH
        return [pltpu.async_copy(adrs[k].at[pl.ds(st, RCH)], I[s * 3 + k],
                                 SI[s * 3 + k]) for k in range(3)]

    idesc = {0: idx_start(0, 0)}
    zdesc = {}
    gdesc = {}
    odesc = {}
    for j in range(nch):
        s = j % 2
        st = base + j * RCH
        for d in idesc[j]:
            d.wait()
        if j >= 2:
            odesc[j - 2].wait()
        zdesc[j] = pltpu.async_copy(z0f.at[pl.ds(st, RCH)], R0[s], SZ[s])
        gdesc[j] = [pltpu.async_copy(zg[k].at[I[s * 3 + k]], RG[s * 3 + k],
                                     SG[s * 3 + k]) for k in range(3)]
        if j >= 1:
            for d in gdesc[j - 1]:
                d.wait()
            zdesc[j - 1].wait()
        if j + 1 < nch:
            idesc[j + 1] = idx_start(j + 1, 1 - s)
        if j >= 1:
            stp = base + (j - 1) * RCH
            row_loop(1 - s)
            odesc[j - 1] = pltpu.async_copy(R0[1 - s],
                                            bfh.at[pl.ds(stp, RCH)],
                                            SO[1 - s])
            pltpu.sync_copy(PRC, prh.at[pl.ds(stp, RCH)])
    sl = (nch - 1) % 2
    stl = base + (nch - 1) * RCH
    for d in gdesc[nch - 1]:
        d.wait()
    zdesc[nch - 1].wait()
    row_loop(sl)
    odesc[nch - 1] = pltpu.async_copy(R0[sl], bfh.at[pl.ds(stl, RCH)],
                                      SO[sl])
    pltpu.sync_copy(PRC, prh.at[pl.ds(stl, RCH)])
    odesc[nch - 2].wait()
    odesc[nch - 1].wait()
    plsc.subcore_barrier()

    bn = sch.shape[0]
    active = (sx == 0) & (cx < bn)
    b = jnp.minimum(cx, bn - 1)

    @pl.when(active)
    def _():
        pltpu.sync_copy(a0q.at[b], A0)
        pltpu.sync_copy(a1q.at[b], A1)
        pltpu.sync_copy(a2q.at[b], A2)
        pltpu.sync_copy(prh.at[pl.ds(b * FQ, FQ)], PR)
        pltpu.sync_copy(initf.at[pl.ds(b * FQ, FQ)], SCR)
        pltpu.sync_copy(qih.at[b], QU.at[pl.ds(0, FQ)])
        pltpu.sync_copy(dih.at[b], DQ)
        pltpu.sync_copy(tgh, TG)

    def step(carry):
        base_q, hi, tl, level, tok = carry
        new_lvl = base_q >= hi
        level = jnp.where(new_lvl, level + 1, level)
        base_q = jnp.where(new_lvl, hi, base_q)
        hi = jnp.where(new_lvl, tl, hi)
        lanes = base_q + iota16
        m = lanes < hi
        fv = plsc.load_gather(QU, [jnp.where(m, lanes, 0)])
        f = jnp.where(m, fv, 0)
        a0v = plsc.load_gather(A0, [f])
        a1v = plsc.load_gather(A1, [f])
        a2v = plsc.load_gather(A2, [f])
        d0 = plsc.load_gather(DQ, [a0v])
        d1 = plsc.load_gather(DQ, [a1v])
        d2 = plsc.load_gather(DQ, [a2v])
        s0 = plsc.load_gather(SCR, [a0v])
        s1 = plsc.load_gather(SCR, [a1v])
        s2 = plsc.load_gather(SCR, [a2v])
        v0 = d0 < level
        v1 = d1 < level
        v2 = d2 < level
        neg = jnp.float32(-1e30)
        nb = jnp.maximum(jnp.maximum(jnp.where(v0, s0, neg),
                                     jnp.where(v1, s1, neg)),
                         jnp.where(v2, s2, neg))
        has = v0 | v1 | v2
        nbv = jnp.where(has, nb, jnp.float32(1.0))
        pf = plsc.load_gather(PR, [f])
        sf = plsc.load_gather(SCR, [f])
        val = jnp.minimum(jnp.maximum(pf, sf), nbv)
        plsc.store_scatter(SCR, [f], val, mask=m)
        dnew = jnp.zeros((16,), jnp.int32) + (level + 1)

        def expand(av, dv, tl, tok):
            cand = m & (dv == INF)
            tokv = tok + iota16
            plsc.store_scatter(TG, [av], tokv, mask=cand)
            tt = plsc.load_gather(TG, [av])
            win = cand & (tt == tokv)
            plsc.store_scatter(DQ, [av], dnew, mask=cand)
            plsc.store_compressed(QU.at[pl.ds(tl, 16)], av, mask=win)
            cnt = plsc.all_reduce_population_count(win)[0]
            return tl + cnt, tok + 16

        tl, tok = expand(a0v, d0, tl, tok)
        tl, tok = expand(a1v, d1, tl, tok)
        tl, tok = expand(a2v, d2, tl, tok)
        return base_q + 16, hi, tl, level, tok

    def not_done(carry):
        base_q, hi, tl, _, _ = carry
        return (base_q < hi) | (hi < tl)

    one_if = jnp.where(active, jnp.int32(1), jnp.int32(0))
    lax.while_loop(not_done, step,
                   (jnp.int32(0), one_if, one_if, jnp.int32(0),
                    jnp.int32(0)))

    @pl.when(active)
    def _():
        pltpu.sync_copy(SCR, sch.at[b])
        pltpu.sync_copy(DQ, dph.at[b])


def _mega(z0f, z1f, z2f, z3f, a0f, a1f, a2f, a0q, a1q, a2q,
          initf, qi, di, tgi, wmh):
    bn = a0q.shape[0]
    c = z0f.shape[1]
    mesh = plsc.VectorSubcoreMesh(core_axis_name="c", subcore_axis_name="s")
    run = functools.partial(
        pl.kernel,
        out_type=[jax.ShapeDtypeStruct((bn * FQ, c), jnp.float32),
                  jax.ShapeDtypeStruct((bn * FQ,), jnp.float32),
                  jax.ShapeDtypeStruct((bn, FQ), jnp.float32),
                  jax.ShapeDtypeStruct((bn, FQ), jnp.int32)],
        mesh=mesh,
        scratch_types=[pltpu.VMEM((RCH,), jnp.int32)] * 6
        + [pltpu.VMEM((RCH, c), jnp.float32)] * 8
        + [pltpu.VMEM((RCH,), jnp.float32)]
        + [pltpu.VMEM((144,), jnp.float32)]
        + [pltpu.VMEM((FQ,), jnp.int32)] * 3
        + [pltpu.VMEM((FQ,), jnp.float32)] * 2
        + [pltpu.VMEM((FQ,), jnp.int32),
           pltpu.VMEM((FQ + 16,), jnp.int32),
           pltpu.VMEM((FQ,), jnp.int32)]
        + [pltpu.SemaphoreType.DMA] * 16,
        compiler_params=pltpu.CompilerParams(needs_layout_passes=False),
    )(_mega_body)
    return run(z0f, z1f, z2f, z3f, a0f, a1f, a2f, a0q, a1q, a2q,
               initf, qi, di, tgi, wmh)


def _k5_body(feats_ref, bf_ref, depth_ref, out_ref):
    reached = depth_ref[...] != INF
    out_ref[...] = jnp.where(reached, bf_ref[...], feats_ref[...])


def _k5(feats, bf3, depth3):
    bn, fn, c = feats.shape
    grid = (bn, fn // ROWS)
    fspec = pl.BlockSpec((1, ROWS, c), lambda b, j: (b, j, 0))
    return pl.pallas_call(
        _k5_body,
        grid=grid,
        in_specs=[fspec, fspec,
                  pl.BlockSpec((1, ROWS, 1), lambda b, j: (b, j, 0))],
        out_specs=fspec,
        out_shape=jax.ShapeDtypeStruct((bn, fn, c), jnp.float32),
    )(feats, bf3, depth3)


def kernel(x, face_adj, anchors, Wc, bc, Wm, bm):
    bn, cn, fn = x.shape

    feats = jnp.transpose(x, (0, 2, 1))

    Wc4 = jnp.concatenate([Wc[k * cn:(k + 1) * cn] for k in range(4)],
                          axis=1)
    bc2 = bc.reshape(1, cn)
    wmh = jnp.concatenate(
        [Wm.reshape(cn), jnp.full((16,), bm[0], jnp.float32)])

    padq = ((0, 0), (0, FQ - fn))
    a0q = jnp.pad(face_adj[:, :, 0], padq)
    a1q = jnp.pad(face_adj[:, :, 1], padq)
    a2q = jnp.pad(face_adj[:, :, 2], padq)
    offs = (jnp.arange(bn, dtype=jnp.int32) * FQ)[:, None]
    a0f = (a0q + offs).reshape(bn * FQ)
    a1f = (a1q + offs).reshape(bn * FQ)
    a2f = (a2q + offs).reshape(bn * FQ)

    qi = jnp.zeros((bn, FQ), jnp.int32).at[:, 0].set(anchors)
    di = jnp.full((bn, FQ), INF, jnp.int32).at[
        jnp.arange(bn), anchors].set(-1)
    tgi = jnp.full((FQ,), -1, jnp.int32)

    wm2 = Wm.reshape(1, cn)
    bm2 = bm.reshape(1, 1)
    z0, z1, z2, z3, initsc = _k1(feats, Wc4, bc2, wm2, bm2)
    initf = initsc.reshape(bn * FQ)
    z0f = z0.reshape(bn * FQ, cn)
    z1f = z1.reshape(bn * FQ, cn)
    z2f = z2.reshape(bn * FQ, cn)
    z3f = z3.reshape(bn * FQ, cn)
    bf, pred, scores, depth = _mega(z0f, z1f, z2f, z3f, a0f, a1f, a2f,
                                    a0q, a1q, a2q, initf, qi, di, tgi, wmh)
    bf3 = bf.reshape(bn, FQ, cn)
    depth3 = depth.reshape(bn, FQ, 1)
    final_features = _k5(feats, bf3, depth3)
    final_scores = scores[:, :fn].reshape(bn, fn, 1)
    return final_features, final_scores
